# Initial kernel scaffold; baseline (speedup 1.0000x reference)
import jax
import jax.numpy as jnp
from jax.experimental import pallas as pl


def kernel(query, support, q_l1, q_deg_l, q_r1, q_deg_r, s_l1, s_deg_l, s_r1, s_deg_r, symbol_emb, proj_w_W, proj_w_b, proj_b, gate_w_W, gate_w_b, gate_b, se_W1, se_b1, se_W2, se_b2, ln_g, ln_b, lstm_Wih, lstm_Whh, lstm_bih, lstm_bhh):
    raise NotImplementedError("write your pallas kernel here")


# speedup vs baseline: 5.1487x; 5.1487x over previous
import functools

import jax
import jax.numpy as jnp
from jax import lax
from jax.experimental import pallas as pl
from jax.experimental.pallas import tpu as pltpu
from jax.experimental.pallas import tpu_sc as plsc

_SC_NC = 2
_SC_NS = 16
_NW = _SC_NC * _SC_NS
_C = 128
_CS = 88


def _sc_body(per_w, n_chunks, per_w_s, n_chunks_s, nd,
             emb_h, rel_h, ent_h, sid_h, prod_h, selfs_h,
             idx_a, idx_b, a_v, b_v, s_idx, s_v, sem_a, sem_b):
    wid = lax.axis_index("s") * _SC_NC + lax.axis_index("c")
    base = wid * per_w

    def chunk(i, carry):
        off = base + i * _C
        pltpu.sync_copy(rel_h.at[pl.ds(off, _C)], idx_a)
        pltpu.sync_copy(ent_h.at[pl.ds(off, _C)], idx_b)
        ca = pltpu.async_copy(emb_h.at[idx_a], a_v, sem_a)
        cb = pltpu.async_copy(emb_h.at[idx_b], b_v, sem_b)
        ca.wait()
        cb.wait()

        def row(r, c2):
            for j in range(nd):
                s = pl.ds(j * 16, 16)
                a_v[r, s] = a_v[r, s] * b_v[r, s]
            return c2

        lax.fori_loop(0, _C, row, 0)
        pltpu.sync_copy(a_v, prod_h.at[pl.ds(off, _C)])
        return carry

    lax.fori_loop(0, n_chunks, chunk, 0)

    base_s = wid * per_w_s

    def schunk(i, carry):
        off = base_s + i * _CS
        pltpu.sync_copy(sid_h.at[pl.ds(off, _CS)], s_idx)
        pltpu.async_copy(emb_h.at[s_idx], s_v, sem_a).wait()
        pltpu.sync_copy(s_v, selfs_h.at[pl.ds(off, _CS)])
        return carry

    lax.fori_loop(0, n_chunks_s, schunk, 0)


def _sc_gather_prod(emb, rel, ent, sid):
    n, d = rel.shape[0], emb.shape[1]
    per_w = n // _NW
    n_chunks = per_w // _C
    ns = sid.shape[0]
    per_w_s = ns // _NW
    n_chunks_s = per_w_s // _CS
    body = functools.partial(_sc_body, per_w, n_chunks, per_w_s, n_chunks_s,
                             d // 16)
    f = pl.kernel(
        body,
        out_type=(jax.ShapeDtypeStruct((n, d), jnp.float32),
                  jax.ShapeDtypeStruct((ns, d), jnp.float32)),
        mesh=plsc.VectorSubcoreMesh(core_axis_name="c", subcore_axis_name="s",
                                    num_cores=_SC_NC, num_subcores=_SC_NS),
        scratch_types=[
            pltpu.VMEM((_C,), jnp.int32),
            pltpu.VMEM((_C,), jnp.int32),
            pltpu.VMEM((_C, d), jnp.float32),
            pltpu.VMEM((_C, d), jnp.float32),
            pltpu.VMEM((_CS,), jnp.int32),
            pltpu.VMEM((_CS, d), jnp.float32),
            pltpu.SemaphoreType.DMA,
            pltpu.SemaphoreType.DMA,
        ])
    return f(emb, rel, ent, sid)


def _nbr_body(pad, prod_ref, rels_ref, selfs_ref, pw_ref, pb_ref,
              gwl_ref, gwr_ref, gb_ref, out_ref):
    t, maxn, d = prod_ref.shape
    x = prod_ref[...].reshape(t * maxn, d)
    nv = jnp.dot(x, pw_ref[...], preferred_element_type=jnp.float32) + pb_ref[...]
    nv = jnp.where(nv >= 0, nv, 0.01 * nv)
    mask = (rels_ref[...] != pad).astype(jnp.float32)
    nv = nv.reshape(t, maxn, d) * mask[:, :, None]
    cnt = jnp.sum(mask, axis=1, keepdims=True)
    agg = jnp.sum(nv, axis=1) / (cnt + 1e-9)
    se = selfs_ref[...]
    logit = (jnp.sum(se * gwl_ref[...], axis=1, keepdims=True)
             + jnp.sum(agg * gwr_ref[...], axis=1, keepdims=True)
             + gb_ref[...])
    gate = jax.nn.sigmoid(logit)
    out_ref[...] = jnp.tanh(se + gate * agg)


def _senc(x, w1, b1, w2, b2, g, b):
    h = jnp.maximum(jnp.dot(x, w1, preferred_element_type=jnp.float32) + b1, 0.0)
    z = jnp.dot(h, w2, preferred_element_type=jnp.float32) + b2 + x
    mu = jnp.mean(z, axis=-1, keepdims=True)
    zc = z - mu
    var = jnp.sum(zc * zc, axis=-1, keepdims=True) / (z.shape[-1] - 1)
    sd = jnp.sqrt(var)
    return zc / (sd + 1e-3) * g + b


def _sup_body(l_ref, r_ref, w1_ref, b1_ref, w2_ref, b2_ref, g_ref, bl_ref,
              out_ref):
    x = jnp.concatenate([l_ref[...], r_ref[...]], axis=1)
    y = _senc(x, w1_ref[...], b1_ref[...], w2_ref[...], b2_ref[...],
              g_ref[...], bl_ref[...])
    m = jnp.mean(y, axis=0, keepdims=True)
    out_ref[...] = jnp.broadcast_to(m, out_ref.shape)


def _q_body(l_ref, r_ref, sup_ref, w1_ref, b1_ref, w2_ref, b2_ref, g_ref,
            bl_ref, wih_ref, bsum_ref, whhh_ref, whhr_ref, out_ref):
    x = jnp.concatenate([l_ref[...], r_ref[...]], axis=1)
    dm = x.shape[1]
    qry = _senc(x, w1_ref[...], b1_ref[...], w2_ref[...], b2_ref[...],
                g_ref[...], bl_ref[...])
    sup = sup_ref[0:1, :]
    base = jnp.dot(qry, wih_ref[...], preferred_element_type=jnp.float32) \
        + bsum_ref[...]
    sup_t = jnp.dot(sup, whhr_ref[...], preferred_element_type=jnp.float32)
    hid = base.shape[1] // 4

    def split(gs):
        return (gs[:, :hid], gs[:, hid:2 * hid], gs[:, 2 * hid:3 * hid],
                gs[:, 3 * hid:])

    i_, f_, g_, o_ = split(base)
    c = jax.nn.sigmoid(i_) * jnp.tanh(g_)
    h = qry + (jax.nn.sigmoid(o_) * jnp.tanh(c))[:, :dm]
    for _ in range(3):
        gs = base + sup_t + jnp.dot(h, whhh_ref[...],
                                    preferred_element_type=jnp.float32)
        i_, f_, g_, o_ = split(gs)
        c = jax.nn.sigmoid(f_) * c + jax.nn.sigmoid(i_) * jnp.tanh(g_)
        h = qry + (jax.nn.sigmoid(o_) * jnp.tanh(c))[:, :dm]
    out_ref[...] = jnp.sum(h * sup, axis=1, keepdims=True)


def kernel(query, support, q_l1, q_deg_l, q_r1, q_deg_r, s_l1, s_deg_l,
           s_r1, s_deg_r, symbol_emb, proj_w_W, proj_w_b, proj_b, gate_w_W,
           gate_w_b, gate_b, se_W1, se_b1, se_W2, se_b2, ln_g, ln_b,
           lstm_Wih, lstm_Whh, lstm_bih, lstm_bhh):
    b = query.shape[0]
    few = support.shape[0]
    maxn = q_l1.shape[1]
    d = symbol_emb.shape[1]
    pad = symbol_emb.shape[0] - 1
    rows = 2 * b + 2 * few

    rel_ids = jnp.concatenate([
        q_l1[:, :, 0].reshape(-1), q_r1[:, :, 0].reshape(-1),
        s_l1[:, :, 0].reshape(-1), s_r1[:, :, 0].reshape(-1)]).astype(jnp.int32)
    ent_ids = jnp.concatenate([
        q_l1[:, :, 1].reshape(-1), q_r1[:, :, 1].reshape(-1),
        s_l1[:, :, 1].reshape(-1), s_r1[:, :, 1].reshape(-1)]).astype(jnp.int32)
    sgran = _NW * _CS
    ns = ((rows + sgran - 1) // sgran) * sgran
    self_ids = jnp.concatenate([
        query[:, 0], query[:, 1], support[:, 0], support[:, 1],
        jnp.zeros((ns - rows,), query.dtype)]).astype(jnp.int32)

    prod, selfs = _sc_gather_prod(symbol_emb, rel_ids, ent_ids, self_ids)
    prod3 = prod.reshape(rows, maxn, d)
    rels = jnp.concatenate(
        [q_l1[:, :, 0], q_r1[:, :, 0], s_l1[:, :, 0], s_r1[:, :, 0]],
        axis=0).astype(jnp.int32)

    pwT = proj_w_W.T
    pb = (proj_w_b + proj_b).reshape(1, d)
    gwl = gate_w_W[:, :d]
    gwr = gate_w_W[:, d:]
    gb = (gate_w_b + gate_b).reshape(1, 1)

    t1 = 128
    nbr_out = pl.pallas_call(
        functools.partial(_nbr_body, pad),
        grid=(rows // t1,),
        in_specs=[
            pl.BlockSpec((t1, maxn, d), lambda i: (i, 0, 0)),
            pl.BlockSpec((t1, maxn), lambda i: (i, 0)),
            pl.BlockSpec((t1, d), lambda i: (i, 0)),
            pl.BlockSpec((d, d), lambda i: (0, 0)),
            pl.BlockSpec((1, d), lambda i: (0, 0)),
            pl.BlockSpec((1, d), lambda i: (0, 0)),
            pl.BlockSpec((1, d), lambda i: (0, 0)),
            pl.BlockSpec((1, 1), lambda i: (0, 0)),
        ],
        out_specs=pl.BlockSpec((t1, d), lambda i: (i, 0)),
        out_shape=jax.ShapeDtypeStruct((rows, d), jnp.float32),
    )(prod3, rels, selfs[:rows], pwT, pb, gwl, gwr, gb)

    dm = 2 * d
    w1T = se_W1.T
    b1 = se_b1.reshape(1, -1)
    w2T = se_W2.T
    b2 = se_b2.reshape(1, -1)
    lng = ln_g.reshape(1, -1)
    lnb = ln_b.reshape(1, -1)

    sbase = (2 * b) // few
    sup = pl.pallas_call(
        _sup_body,
        grid=(1,),
        in_specs=[
            pl.BlockSpec((few, d), lambda i: (sbase, 0)),
            pl.BlockSpec((few, d), lambda i: (sbase + 1, 0)),
            pl.BlockSpec(w1T.shape, lambda i: (0, 0)),
            pl.BlockSpec(b1.shape, lambda i: (0, 0)),
            pl.BlockSpec(w2T.shape, lambda i: (0, 0)),
            pl.BlockSpec(b2.shape, lambda i: (0, 0)),
            pl.BlockSpec(lng.shape, lambda i: (0, 0)),
            pl.BlockSpec(lnb.shape, lambda i: (0, 0)),
        ],
        out_specs=pl.BlockSpec((8, dm), lambda i: (0, 0)),
        out_shape=jax.ShapeDtypeStruct((8, dm), jnp.float32),
    )(nbr_out, nbr_out, w1T, b1, w2T, b2, lng, lnb)

    wihT = lstm_Wih.T
    bsum = (lstm_bih + lstm_bhh).reshape(1, -1)
    whhhT = lstm_Whh[:, :dm].T
    whhrT = lstm_Whh[:, dm:].T

    t2 = 256
    nblk = b // t2
    scores = pl.pallas_call(
        _q_body,
        grid=(nblk,),
        in_specs=[
            pl.BlockSpec((t2, d), lambda i: (i, 0)),
            pl.BlockSpec((t2, d), lambda i: (i + nblk, 0)),
            pl.BlockSpec((8, dm), lambda i: (0, 0)),
            pl.BlockSpec(w1T.shape, lambda i: (0, 0)),
            pl.BlockSpec(b1.shape, lambda i: (0, 0)),
            pl.BlockSpec(w2T.shape, lambda i: (0, 0)),
            pl.BlockSpec(b2.shape, lambda i: (0, 0)),
            pl.BlockSpec(lng.shape, lambda i: (0, 0)),
            pl.BlockSpec(lnb.shape, lambda i: (0, 0)),
            pl.BlockSpec(wihT.shape, lambda i: (0, 0)),
            pl.BlockSpec(bsum.shape, lambda i: (0, 0)),
            pl.BlockSpec(whhhT.shape, lambda i: (0, 0)),
            pl.BlockSpec(whhrT.shape, lambda i: (0, 0)),
        ],
        out_specs=pl.BlockSpec((t2, 1), lambda i: (i, 0)),
        out_shape=jax.ShapeDtypeStruct((b, 1), jnp.float32),
    )(nbr_out, nbr_out, sup, w1T, b1, w2T, b2, lng, lnb, wihT, bsum,
      whhhT, whhrT)

    return scores[:, 0]
